# Initial kernel scaffold; baseline (speedup 1.0000x reference)
import jax
import jax.numpy as jnp
from jax.experimental import pallas as pl


def kernel(x, label_embedding_weight):
    raise NotImplementedError("write your pallas kernel here")


# speedup vs baseline: 3.9380x; 3.9380x over previous
import functools

import jax
import jax.numpy as jnp
from jax import lax
from jax.experimental import pallas as pl
from jax.experimental.pallas import tpu as pltpu
from jax.experimental.pallas import tpu_sc as plsc

CHUNK = 512


@functools.partial(jax.jit, static_argnums=(2, 3))
def _sc_gather(idx, table, B, D):
    info = plsc.get_sparse_core_info()
    NC, NS = info.num_cores, info.num_subcores
    NW = NC * NS
    b_per_w = B // NW
    n_chunks = b_per_w // CHUNK
    mesh = plsc.VectorSubcoreMesh(core_axis_name="c", subcore_axis_name="s")

    @functools.partial(
        pl.kernel,
        out_type=jax.ShapeDtypeStruct((B, D), jnp.float32),
        mesh=mesh,
        scratch_types=[
            pltpu.VMEM((CHUNK,), jnp.int32),
            pltpu.VMEM((CHUNK, D), jnp.float32),
            pltpu.SemaphoreType.DMA,
        ],
        compiler_params=pltpu.CompilerParams(use_tc_tiling_on_sc=False),
    )
    def k(idx_hbm, table_hbm, out_hbm, idx_v, rows_v, sem):
        wid = lax.axis_index("s") * NC + lax.axis_index("c")
        base = wid * b_per_w

        def body(i, carry):
            off = base + i * CHUNK
            pltpu.sync_copy(idx_hbm.at[pl.ds(off, CHUNK)], idx_v)
            pltpu.async_copy(table_hbm.at[idx_v], rows_v, sem).wait()
            pltpu.sync_copy(rows_v, out_hbm.at[pl.ds(off, CHUNK)])
            return carry

        lax.fori_loop(0, n_chunks, body, 0)

    return k(idx, table)


def kernel(x, label_embedding_weight):
    B0, B1, _ = x.shape
    B = B0 * B1
    D = label_embedding_weight.shape[1]
    idx = x[:, :, 1].astype(jnp.int32).reshape(B)
    out = _sc_gather(idx, label_embedding_weight, B, D)
    return out.reshape(B0, B1, D)


# speedup vs baseline: 4.1886x; 1.0636x over previous
import functools

import jax
import jax.numpy as jnp
from jax import lax
from jax.experimental import pallas as pl
from jax.experimental.pallas import tpu as pltpu
from jax.experimental.pallas import tpu_sc as plsc

CHUNK = 800
NBUF = 2


@functools.partial(jax.jit, static_argnums=(2, 3))
def _sc_gather(idx, table, B, D):
    info = plsc.get_sparse_core_info()
    NC, NS = info.num_cores, info.num_subcores
    NW = NC * NS
    b_per_w = B // NW
    n_chunks = b_per_w // CHUNK
    mesh = plsc.VectorSubcoreMesh(core_axis_name="c", subcore_axis_name="s")

    @functools.partial(
        pl.kernel,
        out_type=jax.ShapeDtypeStruct((B, D), jnp.float32),
        mesh=mesh,
        scratch_types=[
            pltpu.VMEM((NBUF, CHUNK), jnp.int32),
            pltpu.VMEM((NBUF, CHUNK, D), jnp.float32),
            pltpu.SemaphoreType.DMA,
            pltpu.SemaphoreType.DMA,
        ],
        compiler_params=pltpu.CompilerParams(use_tc_tiling_on_sc=False),
    )
    def k(idx_hbm, table_hbm, out_hbm, idx_v, rows_v, gsem, wsem):
        wid = lax.axis_index("s") * NC + lax.axis_index("c")
        base = wid * b_per_w

        def issue_gather(i, slot):
            off = base + i * CHUNK
            pltpu.sync_copy(idx_hbm.at[pl.ds(off, CHUNK)], idx_v.at[slot])
            pltpu.async_copy(table_hbm.at[idx_v.at[slot]], rows_v.at[slot], gsem)

        def wait_write():
            pltpu.make_async_copy(
                rows_v.at[0], out_hbm.at[pl.ds(base, CHUNK)], wsem
            ).wait()

        issue_gather(0, 0)

        def body(i, carry):
            slot = lax.rem(i, NBUF)
            nslot = lax.rem(i + 1, NBUF)

            @pl.when(i >= 1)
            def _():
                wait_write()

            @pl.when(i + 1 < n_chunks)
            def _():
                issue_gather(i + 1, nslot)

            pltpu.make_async_copy(
                table_hbm.at[idx_v.at[slot]], rows_v.at[slot], gsem
            ).wait()
            off = base + i * CHUNK
            pltpu.async_copy(rows_v.at[slot], out_hbm.at[pl.ds(off, CHUNK)], wsem)
            return carry

        lax.fori_loop(0, n_chunks, body, 0)
        wait_write()

    return k(idx, table)


def kernel(x, label_embedding_weight):
    B0, B1, _ = x.shape
    B = B0 * B1
    D = label_embedding_weight.shape[1]
    idx = x[:, :, 1].astype(jnp.int32).reshape(B)
    out = _sc_gather(idx, label_embedding_weight, B, D)
    return out.reshape(B0, B1, D)


# speedup vs baseline: 4.2036x; 1.0036x over previous
import functools

import jax
import jax.numpy as jnp
from jax import lax
from jax.experimental import pallas as pl
from jax.experimental.pallas import tpu as pltpu
from jax.experimental.pallas import tpu_sc as plsc

NBUF = 2


@functools.partial(jax.jit, static_argnums=(2, 3, 4))
def _sc_gather(idx, table, B0, B1, D):
    B = B0 * B1
    info = plsc.get_sparse_core_info()
    NC, NS = info.num_cores, info.num_subcores
    NW = NC * NS
    b_per_w = B // NW
    CHUNK = B1
    n_chunks = b_per_w // CHUNK
    mesh = plsc.VectorSubcoreMesh(core_axis_name="c", subcore_axis_name="s")

    @functools.partial(
        pl.kernel,
        out_type=jax.ShapeDtypeStruct((B0, B1, D), jnp.float32),
        mesh=mesh,
        scratch_types=[
            pltpu.VMEM((b_per_w,), jnp.int32),
            pltpu.VMEM((NBUF, CHUNK, D), jnp.float32),
            pltpu.SemaphoreType.DMA,
            pltpu.SemaphoreType.DMA,
            pltpu.SemaphoreType.DMA,
            pltpu.SemaphoreType.DMA,
        ],
        compiler_params=pltpu.CompilerParams(use_tc_tiling_on_sc=False),
    )
    def k(idx_hbm, table_hbm, out_hbm, idx_v, rows_v, g0, g1, w0, w1):
        wid = lax.axis_index("s") * NC + lax.axis_index("c")
        base = wid * b_per_w
        row0 = wid * n_chunks
        gsem = (g0, g1)
        wsem = (w0, w1)

        pltpu.sync_copy(idx_hbm.at[pl.ds(base, b_per_w)], idx_v)

        def issue_gather(i, b):
            pltpu.async_copy(
                table_hbm.at[idx_v.at[pl.ds(i * CHUNK, CHUNK)]],
                rows_v.at[b],
                gsem[b],
            )

        def wait_write(i, b):
            pltpu.make_async_copy(rows_v.at[b], out_hbm.at[row0 + i], wsem[b]).wait()

        issue_gather(0, 0)

        def body(g, carry):
            for b in range(NBUF):
                i = g * NBUF + b
                nb = (b + 1) % NBUF

                @pl.when(i + 1 < n_chunks)
                def _():
                    @pl.when(i >= 1)
                    def _():
                        wait_write(i - 1, nb)

                    issue_gather(i + 1, nb)

                pltpu.make_async_copy(
                    table_hbm.at[idx_v.at[pl.ds(i * CHUNK, CHUNK)]],
                    rows_v.at[b],
                    gsem[b],
                ).wait()
                pltpu.async_copy(rows_v.at[b], out_hbm.at[row0 + i], wsem[b])
            return carry

        lax.fori_loop(0, n_chunks // NBUF, body, 0)
        wait_write(n_chunks - 1, (n_chunks - 1) % NBUF)

    return k(idx, table)


def kernel(x, label_embedding_weight):
    B0, B1, _ = x.shape
    D = label_embedding_weight.shape[1]
    idx = x[:, :, 1].astype(jnp.int32).reshape(B0 * B1)
    return _sc_gather(idx, label_embedding_weight, B0, B1, D)
